# baseline (speedup 1.0000x reference)
import functools

import jax
import jax.numpy as jnp
from jax import lax
from jax.experimental import pallas as pl
from jax.experimental.pallas import tpu as pltpu
from jax.experimental.pallas import tpu_sc as plsc

VOCAB = 30522
D = 768
B = 128
N = 50
L = 32
EPS = 1e-12

LANES = 16
NV = D // LANES
NC = 2
NS = 16
NW = NC * NS
T = B * N * L
TPW = T // NW
CB = L
NCHUNK = TPW // CB
NPAIR = NCHUNK // 2

_MAGIC = 0x5F3759DF


def _lane_sum(x):
    i = lax.iota(jnp.int32, LANES)
    for k in (8, 4, 2, 1):
        x = x + x.at[i ^ k].get(mode="promise_in_bounds")
    return x


def _ln_chunk(rows_ref, out_ref, pos_ref, g_ref, b_ref):
    inv_d = jnp.float32(1.0 / D)
    c15 = jnp.float32(1.5)

    def token_body(t, carry):
        zero = jnp.zeros((LANES,), jnp.float32)
        s = [zero, zero, zero, zero]
        q = [zero, zero, zero, zero]
        xs = []
        for c in range(NV):
            sl = pl.ds(c * LANES, LANES)
            x = rows_ref[t, sl] + pos_ref[t, sl]
            xs.append(x)
            s[c % 4] = s[c % 4] + x
            q[c % 4] = q[c % 4] + x * x
        mean_v = _lane_sum((s[0] + s[1]) + (s[2] + s[3])) * inv_d
        meansq_v = _lane_sum((q[0] + q[1]) + (q[2] + q[3])) * inv_d
        v = meansq_v - mean_v * mean_v + jnp.float32(EPS)
        y = plsc.bitcast(
            jnp.int32(_MAGIC)
            - lax.shift_right_arithmetic(plsc.bitcast(v, jnp.int32), 1),
            jnp.float32)
        hv = jnp.float32(-0.5) * v
        y = y * (c15 + hv * y * y)
        y = y * (c15 + hv * y * y)
        y = y * (c15 + hv * y * y)
        for c in range(NV):
            sl = pl.ds(c * LANES, LANES)
            out_ref[t, sl] = (xs[c] - mean_v) * y
        return carry

    lax.fori_loop(0, CB, token_body, 0)

    def col_body(c, carry):
        sl = pl.ds(c * LANES, LANES)
        g = g_ref[sl]
        b = b_ref[sl]
        for t in range(CB):
            out_ref[t, sl] = out_ref[t, sl] * g + b
        return carry

    lax.fori_loop(0, NV, col_body, 0)


def _make_sc_kernel():
    mesh = plsc.VectorSubcoreMesh(core_axis_name="c", subcore_axis_name="s")

    @functools.partial(
        pl.kernel,
        out_type=jax.ShapeDtypeStruct((T, D), jnp.float32),
        mesh=mesh,
        compiler_params=pltpu.CompilerParams(needs_layout_passes=False),
        scratch_types=[
            pltpu.VMEM((CB,), jnp.int32),
            pltpu.VMEM((CB,), jnp.int32),
            pltpu.VMEM((CB, D), jnp.float32),
            pltpu.VMEM((CB, D), jnp.float32),
            pltpu.VMEM((CB, D), jnp.float32),
            pltpu.VMEM((CB, D), jnp.float32),
            pltpu.VMEM((L, D), jnp.float32),
            pltpu.VMEM((D,), jnp.float32),
            pltpu.VMEM((D,), jnp.float32),
            pltpu.SemaphoreType.DMA,
            pltpu.SemaphoreType.DMA,
            pltpu.SemaphoreType.DMA,
            pltpu.SemaphoreType.DMA,
        ],
    )
    def sc_kernel(idx_hbm, table_hbm, pos_hbm, gamma_hbm, beta_hbm, out_hbm,
                  idx0, idx1, rows0, rows1, outb0, outb1, pos_v, g_v, b_v,
                  gsem0, gsem1, ssem0, ssem1):
        wid = lax.axis_index("s") * NC + lax.axis_index("c")
        base = wid * TPW
        pltpu.sync_copy(pos_hbm, pos_v)
        pltpu.sync_copy(gamma_hbm, g_v)
        pltpu.sync_copy(beta_hbm, b_v)

        def fetch(chunk, idx_ref, rows_ref, sem):
            off = base + chunk * CB
            pltpu.sync_copy(idx_hbm.at[pl.ds(off, CB)], idx_ref)
            pltpu.make_async_copy(table_hbm.at[idx_ref], rows_ref, sem).start()

        def wait_fetch(idx_ref, rows_ref, sem):
            pltpu.make_async_copy(table_hbm.at[idx_ref], rows_ref, sem).wait()

        def store(chunk, out_ref, sem):
            off = base + chunk * CB
            pltpu.make_async_copy(out_ref, out_hbm.at[pl.ds(off, CB)], sem).start()

        def wait_store(out_ref, sem):
            pltpu.make_async_copy(out_ref, out_hbm.at[pl.ds(0, CB)], sem).wait()

        fetch(0, idx0, rows0, gsem0)

        def pair_body(i, carry):
            c0 = 2 * i
            fetch(c0 + 1, idx1, rows1, gsem1)
            wait_fetch(idx0, rows0, gsem0)

            @pl.when(i > 0)
            def _():
                wait_store(outb0, ssem0)

            _ln_chunk(rows0, outb0, pos_v, g_v, b_v)
            store(c0, outb0, ssem0)
            fetch(jnp.minimum(c0 + 2, NCHUNK - 1), idx0, rows0, gsem0)
            wait_fetch(idx1, rows1, gsem1)

            @pl.when(i > 0)
            def _():
                wait_store(outb1, ssem1)

            _ln_chunk(rows1, outb1, pos_v, g_v, b_v)
            store(c0 + 1, outb1, ssem1)
            return carry

        lax.fori_loop(0, NPAIR, pair_body, 0)
        wait_fetch(idx0, rows0, gsem0)
        wait_store(outb0, ssem0)
        wait_store(outb1, ssem1)

    return sc_kernel


_SC_KERNEL = _make_sc_kernel()


def kernel(news_batch, table, pos_emb, gamma, beta):
    idx = news_batch.reshape(T).astype(jnp.int32)
    out = _SC_KERNEL(idx, table, pos_emb, gamma, beta)
    return out.reshape(B, N, L, D)


# speedup vs baseline: 2.5551x; 2.5551x over previous
import functools

import jax
import jax.numpy as jnp
from jax import lax
from jax.experimental import pallas as pl
from jax.experimental.pallas import tpu as pltpu
from jax.experimental.pallas import tpu_sc as plsc

VOCAB = 30522
D = 768
B = 128
N = 50
L = 32
EPS = 1e-12

LANES = 16
NV = D // LANES
NC = 2
NS = 16
NW = NC * NS
T = B * N * L
TPW = T // NW
CB = L
NCHUNK = TPW // CB
NPAIR = NCHUNK // 2

_MAGIC = 0x5F3759DF


def _lane_sum(x):
    i = lax.iota(jnp.int32, LANES)
    for k in (8, 4, 2, 1):
        x = x + x.at[i ^ k].get(mode="promise_in_bounds")
    return x


def _ln_chunk(rows_ref, out_ref, pos_ref, g_ref, b_ref):
    inv_d = jnp.float32(1.0 / D)
    c15 = jnp.float32(1.5)

    @plsc.parallel_loop(0, CB, 1, unroll=2)
    def token_body(t):
        zero = jnp.zeros((LANES,), jnp.float32)
        s = [zero, zero, zero, zero]
        q = [zero, zero, zero, zero]
        for c in range(NV):
            sl = pl.ds(c * LANES, LANES)
            x = rows_ref[t, sl] + pos_ref[t, sl]
            rows_ref[t, sl] = x
            s[c % 4] = s[c % 4] + x
            q[c % 4] = q[c % 4] + x * x
        mean_v = _lane_sum((s[0] + s[1]) + (s[2] + s[3])) * inv_d
        meansq_v = _lane_sum((q[0] + q[1]) + (q[2] + q[3])) * inv_d
        v = meansq_v - mean_v * mean_v + jnp.float32(EPS)
        y = plsc.bitcast(
            jnp.int32(_MAGIC)
            - lax.shift_right_arithmetic(plsc.bitcast(v, jnp.int32), 1),
            jnp.float32)
        hv = jnp.float32(-0.5) * v
        y = y * (c15 + hv * y * y)
        y = y * (c15 + hv * y * y)
        y = y * (c15 + hv * y * y)
        for c in range(NV):
            sl = pl.ds(c * LANES, LANES)
            out_ref[t, sl] = (rows_ref[t, sl] - mean_v) * y

    @plsc.parallel_loop(0, NV, 1)
    def col_body(c):
        sl = pl.ds(c * LANES, LANES)
        g = g_ref[sl]
        b = b_ref[sl]
        for t in range(CB):
            out_ref[t, sl] = out_ref[t, sl] * g + b


def _make_sc_kernel():
    mesh = plsc.VectorSubcoreMesh(core_axis_name="c", subcore_axis_name="s")

    @functools.partial(
        pl.kernel,
        out_type=jax.ShapeDtypeStruct((T, D), jnp.float32),
        mesh=mesh,
        compiler_params=pltpu.CompilerParams(needs_layout_passes=False),
        scratch_types=[
            pltpu.VMEM((CB,), jnp.int32),
            pltpu.VMEM((CB,), jnp.int32),
            pltpu.VMEM((CB, D), jnp.float32),
            pltpu.VMEM((CB, D), jnp.float32),
            pltpu.VMEM((CB, D), jnp.float32),
            pltpu.VMEM((CB, D), jnp.float32),
            pltpu.VMEM((L, D), jnp.float32),
            pltpu.VMEM((D,), jnp.float32),
            pltpu.VMEM((D,), jnp.float32),
            pltpu.SemaphoreType.DMA,
            pltpu.SemaphoreType.DMA,
            pltpu.SemaphoreType.DMA,
            pltpu.SemaphoreType.DMA,
        ],
    )
    def sc_kernel(idx_hbm, table_hbm, pos_hbm, gamma_hbm, beta_hbm, out_hbm,
                  idx0, idx1, rows0, rows1, outb0, outb1, pos_v, g_v, b_v,
                  gsem0, gsem1, ssem0, ssem1):
        wid = lax.axis_index("s") * NC + lax.axis_index("c")
        base = wid * TPW
        pltpu.sync_copy(pos_hbm, pos_v)
        pltpu.sync_copy(gamma_hbm, g_v)
        pltpu.sync_copy(beta_hbm, b_v)

        def fetch(chunk, idx_ref, rows_ref, sem):
            off = base + chunk * CB
            pltpu.sync_copy(idx_hbm.at[pl.ds(off, CB)], idx_ref)
            pltpu.make_async_copy(table_hbm.at[idx_ref], rows_ref, sem).start()

        def wait_fetch(idx_ref, rows_ref, sem):
            pltpu.make_async_copy(table_hbm.at[idx_ref], rows_ref, sem).wait()

        def store(chunk, out_ref, sem):
            off = base + chunk * CB
            pltpu.make_async_copy(out_ref, out_hbm.at[pl.ds(off, CB)], sem).start()

        def wait_store(out_ref, sem):
            pltpu.make_async_copy(out_ref, out_hbm.at[pl.ds(0, CB)], sem).wait()

        fetch(0, idx0, rows0, gsem0)

        def pair_body(i, carry):
            c0 = 2 * i
            fetch(c0 + 1, idx1, rows1, gsem1)
            wait_fetch(idx0, rows0, gsem0)

            @pl.when(i > 0)
            def _():
                wait_store(outb0, ssem0)

            _ln_chunk(rows0, outb0, pos_v, g_v, b_v)
            store(c0, outb0, ssem0)
            fetch(jnp.minimum(c0 + 2, NCHUNK - 1), idx0, rows0, gsem0)
            wait_fetch(idx1, rows1, gsem1)

            @pl.when(i > 0)
            def _():
                wait_store(outb1, ssem1)

            _ln_chunk(rows1, outb1, pos_v, g_v, b_v)
            store(c0 + 1, outb1, ssem1)
            return carry

        lax.fori_loop(0, NPAIR, pair_body, 0)
        wait_fetch(idx0, rows0, gsem0)
        wait_store(outb0, ssem0)
        wait_store(outb1, ssem1)

    return sc_kernel


_SC_KERNEL = _make_sc_kernel()


def kernel(news_batch, table, pos_emb, gamma, beta):
    idx = news_batch.reshape(T).astype(jnp.int32)
    out = _SC_KERNEL(idx, table, pos_emb, gamma, beta)
    return out.reshape(B, N, L, D)
